# baseline (speedup 1.0000x reference)
import jax
import jax.numpy as jnp
from jax.experimental import pallas as pl
from jax.experimental.pallas import tpu as pltpu

_N_BLOCKS = 8


def _fused_gcn(emb_ref, adj_ref, w1_ref, b1_ref, w2_ref, b2_ref, out_ref,
               adj_bf_ref, xw1_ref):
    k = pl.program_id(0)
    n = out_ref.shape[0]
    blk = n // _N_BLOCKS

    adj_bf_ref[pl.ds(k * blk, blk), :] = adj_ref[...].astype(jnp.bfloat16)

    @pl.when(k == 0)
    def _prologue():
        xw1_ref[...] = jnp.dot(emb_ref[...], w1_ref[...],
                               preferred_element_type=jnp.float32)

    @pl.when(k == _N_BLOCKS - 1)
    def _compute():
        adj = adj_bf_ref[...]

        ones_col = jnp.ones((n, 1), dtype=jnp.bfloat16)
        deg = jax.lax.dot_general(
            adj, ones_col, (((0,), (0,)), ((), ())),
            preferred_element_type=jnp.float32) + 1.0
        dinv = jax.lax.rsqrt(deg)

        s1 = xw1_ref[...] * dinv
        t1 = jax.lax.dot_general(
            adj, s1.astype(jnp.bfloat16), (((0,), (0,)), ((), ())),
            preferred_element_type=jnp.float32) + s1
        h1 = jnp.maximum(t1 * dinv + b1_ref[...], 0.0)

        s2 = jnp.dot(h1, w2_ref[...],
                     preferred_element_type=jnp.float32) * dinv
        t2 = jax.lax.dot_general(
            adj, s2.astype(jnp.bfloat16), (((0,), (0,)), ((), ())),
            preferred_element_type=jnp.float32) + s2
        out_ref[...] = t2 * dinv + b2_ref[...]


def kernel(emb, adj, W1, b1, W2, b2):
    n = emb.shape[0]
    hidden = W1.shape[1]
    out_dim = W2.shape[1]
    blk = n // _N_BLOCKS
    full = lambda *_: (0, 0)
    return pl.pallas_call(
        _fused_gcn,
        grid=(_N_BLOCKS,),
        in_specs=[
            pl.BlockSpec((n, W1.shape[0]), full),
            pl.BlockSpec((blk, n), lambda k: (k, 0)),
            pl.BlockSpec((W1.shape[0], hidden), full),
            pl.BlockSpec((1, hidden), full),
            pl.BlockSpec((hidden, out_dim), full),
            pl.BlockSpec((1, out_dim), full),
        ],
        out_specs=pl.BlockSpec((n, out_dim), full),
        out_shape=jax.ShapeDtypeStruct((n, out_dim), jnp.float32),
        scratch_shapes=[
            pltpu.VMEM((n, n), jnp.bfloat16),
            pltpu.VMEM((n, hidden), jnp.float32),
        ],
    )(emb, adj, W1, b1.reshape(1, -1), W2, b2.reshape(1, -1))


# speedup vs baseline: 1.5118x; 1.5118x over previous
import jax
import jax.numpy as jnp
from jax.experimental import pallas as pl


def _fused_gcn(emb_ref, adj_ref, w1_ref, b1_ref, w2_ref, b2_ref, out_ref):
    adj = adj_ref[...].astype(jnp.bfloat16)
    n = adj.shape[0]

    deg_row = jnp.sum(adj_ref[...], axis=0, keepdims=True) + 1.0
    dinv = jnp.transpose(jax.lax.rsqrt(deg_row))

    s1 = jnp.dot(emb_ref[...], w1_ref[...],
                 preferred_element_type=jnp.float32) * dinv
    t1 = jax.lax.dot_general(
        adj, s1.astype(jnp.bfloat16), (((0,), (0,)), ((), ())),
        preferred_element_type=jnp.float32) + s1
    h1 = jnp.maximum(t1 * dinv + b1_ref[...], 0.0)

    s2 = jnp.dot(h1, w2_ref[...], preferred_element_type=jnp.float32) * dinv
    t2 = jax.lax.dot_general(
        adj, s2.astype(jnp.bfloat16), (((0,), (0,)), ((), ())),
        preferred_element_type=jnp.float32) + s2
    out_ref[...] = t2 * dinv + b2_ref[...]


def kernel(emb, adj, W1, b1, W2, b2):
    n = emb.shape[0]
    out_dim = W2.shape[1]
    return pl.pallas_call(
        _fused_gcn,
        out_shape=jax.ShapeDtypeStruct((n, out_dim), jnp.float32),
    )(emb, adj, W1, b1.reshape(1, -1), W2, b2.reshape(1, -1))
